# Initial kernel scaffold; baseline (speedup 1.0000x reference)
import jax
import jax.numpy as jnp
from jax.experimental import pallas as pl


def kernel(x, di):
    raise NotImplementedError("write your pallas kernel here")


# speedup vs baseline: 2.6472x; 2.6472x over previous
import functools

import jax
import jax.numpy as jnp
from jax.experimental import pallas as pl
from jax.experimental.pallas import tpu as pltpu
from jax.experimental.pallas import tpu_sc as plsc

_WINDOW = 128
_ROW = 128


def _gather_sc(idx2d, di, n):
    mesh = plsc.VectorSubcoreMesh(core_axis_name="c", subcore_axis_name="s")

    @functools.partial(
        pl.kernel,
        out_type=jax.ShapeDtypeStruct((n, _ROW), di.dtype),
        mesh=mesh,
    )
    def run(table_hbm, i_hbm, o_hbm):
        def body(i_vmem, o_vmem):
            pltpu.sync_copy(table_hbm.at[i_vmem.at[0]], o_vmem)

        pltpu.emit_pipeline(
            body,
            grid=(n // _WINDOW,),
            in_specs=[pl.BlockSpec((1, _WINDOW), lambda i: (0, i))],
            out_specs=[pl.BlockSpec((_WINDOW, _ROW), lambda i: (i, 0))],
            core_axis_name=("c", "s"),
            dimension_semantics=(pltpu.PARALLEL,),
        )(i_hbm, o_hbm)

    return run(di, idx2d)


def kernel(x, di):
    b, t = x.shape
    n = b * t
    out = _gather_sc(x.reshape(1, n), di, n)
    return out.reshape(b, t, di.shape[1])


# speedup vs baseline: 14.7575x; 5.5747x over previous
import functools

import jax
import jax.numpy as jnp
from jax.experimental import pallas as pl
from jax.experimental.pallas import tpu as pltpu
from jax.experimental.pallas import tpu_sc as plsc

_WINDOW = 128
_ROW = 128


def _gather_sc(idx2d, di, n):
    mesh = plsc.VectorSubcoreMesh(core_axis_name="c", subcore_axis_name="s")

    @functools.partial(
        pl.kernel,
        out_type=jax.ShapeDtypeStruct((n, _ROW), di.dtype),
        mesh=mesh,
        scratch_types=[pltpu.VMEM_SHARED((_ROW, _ROW), di.dtype)],
    )
    def run(table_hbm, i_hbm, o_hbm, table_shared):
        @pl.when(jax.lax.axis_index("s") == 0)
        def _():
            pltpu.sync_copy(table_hbm, table_shared)

        plsc.subcore_barrier()

        def body(i_vmem, o_vmem):
            pltpu.sync_copy(table_shared.at[i_vmem.at[0]], o_vmem)

        pltpu.emit_pipeline(
            body,
            grid=(n // _WINDOW,),
            in_specs=[pl.BlockSpec((1, _WINDOW), lambda i: (0, i))],
            out_specs=[pl.BlockSpec((_WINDOW, _ROW), lambda i: (i, 0))],
            core_axis_name=("c", "s"),
            dimension_semantics=(pltpu.PARALLEL,),
        )(i_hbm, o_hbm)

    return run(di, idx2d)


def kernel(x, di):
    b, t = x.shape
    n = b * t
    out = _gather_sc(x.reshape(1, n), di, n)
    return out.reshape(b, t, di.shape[1])


# speedup vs baseline: 15.3890x; 1.0428x over previous
import functools

import jax
from jax import lax
import jax.numpy as jnp
from jax.experimental import pallas as pl
from jax.experimental.pallas import tpu as pltpu
from jax.experimental.pallas import tpu_sc as plsc

_WINDOW = 128
_K = 2
_ROW = 128


def _gather_sc(idx3d, di, n):
    steps = n // (_K * _WINDOW)
    mesh = plsc.VectorSubcoreMesh(core_axis_name="c", subcore_axis_name="s")

    @functools.partial(
        pl.kernel,
        out_type=jax.ShapeDtypeStruct((n, _ROW), di.dtype),
        mesh=mesh,
        scratch_types=[
            pltpu.VMEM_SHARED((_ROW, _ROW), di.dtype),
            pltpu.SemaphoreType.DMA,
        ],
    )
    def run(table_hbm, i_hbm, o_hbm, table_shared, sem):
        @pl.when(lax.axis_index("s") == 0)
        def _():
            pltpu.sync_copy(table_hbm, table_shared)

        plsc.subcore_barrier()

        def body(i_vmem, o_vmem):
            cps = [
                pltpu.async_copy(
                    table_shared.at[i_vmem.at[0, j]],
                    o_vmem.at[pl.ds(j * _WINDOW, _WINDOW)],
                    sem,
                )
                for j in range(_K)
            ]
            for cp in cps:
                cp.wait()

        pltpu.emit_pipeline(
            body,
            grid=(steps,),
            in_specs=[pl.BlockSpec((1, _K, _WINDOW), lambda i: (i, 0, 0))],
            out_specs=[pl.BlockSpec((_K * _WINDOW, _ROW), lambda i: (i, 0))],
            core_axis_name=("c", "s"),
            dimension_semantics=(pltpu.PARALLEL,),
        )(i_hbm, o_hbm)

    return run(di, idx3d)


def kernel(x, di):
    b, t = x.shape
    n = b * t
    idx3d = x.reshape(n // (_K * _WINDOW), _K, _WINDOW)
    out = _gather_sc(idx3d, di, n)
    return out.reshape(b, t, di.shape[1])
